# baseline (speedup 1.0000x reference)
import functools

import jax
import jax.numpy as jnp
from jax import lax
from jax.experimental import pallas as pl
from jax.experimental.pallas import tpu as pltpu
from jax.experimental.pallas import tpu_sc as plsc

BLK = 256
NC, NS = 2, 16
NW = NC * NS
CH = 32


def _router_meta_body(x_ref, wr_ref, pos_ref, w01_ref, be_ref, aux_ref,
                      *, n_tokens, n_experts, nb_tot):
    x = x_ref[...]
    logits = lax.dot_general(
        x, wr_ref[...], (((1,), (1,)), ((), ())),
        preferred_element_type=jnp.float32)

    e_iota = lax.broadcasted_iota(jnp.int32, logits.shape, 1)
    m1 = jnp.max(logits, axis=-1, keepdims=True)
    i1 = jnp.min(jnp.where(logits == m1, e_iota, n_experts), axis=-1,
                 keepdims=True)
    oh1 = (e_iota == i1).astype(jnp.float32)
    masked = jnp.where(e_iota == i1, -jnp.inf, logits)
    m2 = jnp.max(masked, axis=-1, keepdims=True)
    i2 = jnp.min(jnp.where(masked == m2, e_iota, n_experts), axis=-1,
                 keepdims=True)
    oh2 = (e_iota == i2).astype(jnp.float32)
    w2 = 1.0 / (1.0 + jnp.exp(m1 - m2))
    w1 = 1.0 - w2
    w01_ref[...] = jnp.concatenate([w1, w2], axis=1)

    cnt = oh1 + oh2
    c = cnt
    s = 1
    while s < n_tokens:
        c = c + jnp.concatenate(
            [jnp.zeros((s, n_experts), jnp.float32), c[:-s, :]], axis=0)
        s *= 2
    c_excl = c - cnt
    counts = c[n_tokens - 1:n_tokens, :]
    ci = counts.astype(jnp.int32)
    ca = ((ci + (BLK - 1)) // BLK) * BLK
    off = ca
    s = 1
    while s < n_experts:
        off = off + jnp.concatenate(
            [jnp.zeros((1, s), jnp.int32), off[:, :-s]], axis=1)
        s *= 2
    off_excl_f = (off - ca).astype(jnp.float32)

    slot = off_excl_f + c_excl
    p0 = jnp.sum(oh1 * slot, axis=1, keepdims=True)
    p1 = jnp.sum(oh2 * slot, axis=1, keepdims=True)
    pos_ref[...] = jnp.concatenate([p0, p1], axis=1).astype(jnp.int32)

    bstart = lax.broadcasted_iota(jnp.int32, (nb_tot, n_experts), 0) * BLK
    be_ref[...] = jnp.sum(
        (jnp.broadcast_to(off, (nb_tot, n_experts)) <= bstart
         ).astype(jnp.int32), axis=1, keepdims=True)

    z = jnp.exp(logits - m1)
    sc = z / jnp.sum(z, axis=-1, keepdims=True)
    pi_sum = jnp.sum(sc, axis=0, keepdims=True)
    aux_ref[...] = (jnp.sum(pi_sum * counts)
                    / float(n_tokens * n_tokens)).reshape(1, 1)


def _gmm_body(be_ref, xp_ref, w_ref, y_ref, *, n_experts):
    @pl.when(be_ref[pl.program_id(0)] < n_experts)
    def _():
        y_ref[...] = lax.dot_general(
            xp_ref[...].astype(jnp.bfloat16), w_ref[0],
            (((1,), (1,)), ((), ())), preferred_element_type=jnp.float32)


def _final_body(x_ref, ws_ref, y0_ref, y1_ref, w01_ref, out_ref):
    xb = x_ref[...].astype(jnp.bfloat16)
    ws = (ws_ref[0].astype(jnp.float32)
          + ws_ref[1].astype(jnp.float32)).astype(jnp.bfloat16)
    acc = lax.dot_general(xb, ws, (((1,), (1,)), ((), ())),
                          preferred_element_type=jnp.float32)
    w01 = w01_ref[...]
    acc = acc + w01[:, 0:1] * y0_ref[...] + w01[:, 1:2] * y1_ref[...]
    out_ref[...] = acc


def _make_dispatch(n_tokens, d, nk_pad):
    tpw = n_tokens // NW
    nch = tpw // CH
    mesh = plsc.VectorSubcoreMesh(core_axis_name="c", subcore_axis_name="s")

    @functools.partial(
        pl.kernel, mesh=mesh,
        out_type=jax.ShapeDtypeStruct((nk_pad, d), jnp.float32),
        scratch_types=[
            pltpu.VMEM((2, nch, CH), jnp.int32),
            pltpu.VMEM((CH, d), jnp.float32),
            pltpu.SemaphoreType.DMA,
        ],
    )
    def dispatch(x_hbm, pos_hbm, xp_hbm, idx_v, buf_v, sem):
        wid = lax.axis_index("s") * NC + lax.axis_index("c")
        base = wid * tpw
        pltpu.sync_copy(pos_hbm.at[wid], idx_v)
        for c in range(nch):
            pltpu.sync_copy(x_hbm.at[pl.ds(base + c * CH, CH)], buf_v)
            pltpu.async_copy(buf_v, xp_hbm.at[idx_v.at[0, c]], sem).wait()
            pltpu.async_copy(buf_v, xp_hbm.at[idx_v.at[1, c]], sem).wait()

    return dispatch


def _make_combine(n_tokens, d, nk_pad):
    tpw = n_tokens // NW
    nch = tpw // CH
    mesh = plsc.VectorSubcoreMesh(core_axis_name="c", subcore_axis_name="s")

    @functools.partial(
        pl.kernel, mesh=mesh,
        out_type=(jax.ShapeDtypeStruct((n_tokens, d), jnp.float32),
                  jax.ShapeDtypeStruct((n_tokens, d), jnp.float32)),
        scratch_types=[
            pltpu.VMEM((2, nch, CH), jnp.int32),
            pltpu.VMEM((CH, d), jnp.float32),
            pltpu.SemaphoreType.DMA,
        ],
    )
    def combine(y_hbm, pos_hbm, y0_hbm, y1_hbm, idx_v, buf_v, sem):
        wid = lax.axis_index("s") * NC + lax.axis_index("c")
        base = wid * tpw
        pltpu.sync_copy(pos_hbm.at[wid], idx_v)
        for c in range(nch):
            pltpu.async_copy(y_hbm.at[idx_v.at[0, c]], buf_v, sem).wait()
            pltpu.sync_copy(buf_v, y0_hbm.at[pl.ds(base + c * CH, CH)])
            pltpu.async_copy(y_hbm.at[idx_v.at[1, c]], buf_v, sem).wait()
            pltpu.sync_copy(buf_v, y1_hbm.at[pl.ds(base + c * CH, CH)])

    return combine


def _router_meta(x, W_router, nb_tot):
    n_tokens, _ = x.shape
    n_experts = W_router.shape[0]
    return pl.pallas_call(
        functools.partial(_router_meta_body, n_tokens=n_tokens,
                          n_experts=n_experts, nb_tot=nb_tot),
        out_shape=[
            jax.ShapeDtypeStruct((n_tokens, 2), jnp.int32),
            jax.ShapeDtypeStruct((n_tokens, 2), jnp.float32),
            jax.ShapeDtypeStruct((nb_tot, 1), jnp.int32),
            jax.ShapeDtypeStruct((1, 1), jnp.float32),
        ],
    )(x, W_router)


def _gmm(be, x_perm, we, n_experts, d):
    nb_tot = be.shape[0]
    grid_spec = pltpu.PrefetchScalarGridSpec(
        num_scalar_prefetch=1,
        grid=(nb_tot,),
        in_specs=[
            pl.BlockSpec((BLK, d), lambda i, be_r: (i, 0)),
            pl.BlockSpec((1, d, d),
                         lambda i, be_r: (jnp.minimum(be_r[i], n_experts - 1),
                                          0, 0)),
        ],
        out_specs=pl.BlockSpec((BLK, d), lambda i, be_r: (i, 0)),
    )
    return pl.pallas_call(
        functools.partial(_gmm_body, n_experts=n_experts),
        grid_spec=grid_spec,
        out_shape=jax.ShapeDtypeStruct((x_perm.shape[0], d), jnp.float32),
    )(be, x_perm, we)


def _final(x, ws, y0, y1, w01):
    n_tokens, d = x.shape
    blk = 512
    nb = n_tokens // blk
    return pl.pallas_call(
        _final_body,
        grid=(nb,),
        in_specs=[
            pl.BlockSpec((blk, d), lambda i: (i, 0)),
            pl.BlockSpec((2, d, d), lambda i: (0, 0, 0)),
            pl.BlockSpec((blk, d), lambda i: (i, 0)),
            pl.BlockSpec((blk, d), lambda i: (i, 0)),
            pl.BlockSpec((blk, 2), lambda i: (i, 0)),
        ],
        out_specs=pl.BlockSpec((blk, d), lambda i: (i, 0)),
        out_shape=jax.ShapeDtypeStruct((n_tokens, d), jnp.float32),
    )(x, ws, y0, y1, w01)


def kernel(feat, W_router, W_shared, W_experts):
    B, S, d = feat.shape
    N = B * S
    E = W_router.shape[0]
    topk = 2
    nb_tot = (N * topk) // BLK + E
    nk_pad = nb_tot * BLK

    x = feat.reshape(N, d)
    we = W_experts.astype(jnp.bfloat16)
    ws = W_shared.reshape(-1, d, d).astype(jnp.bfloat16)

    pos, w01, be2d, aux = _router_meta(x, W_router, nb_tot)
    tpw = N // NW
    nch = tpw // CH
    pos_sc = pos.T.reshape(2, NW, nch, CH).transpose(1, 0, 2, 3)
    be = be2d.reshape(nb_tot)

    x_perm = _make_dispatch(N, d, nk_pad)(x, pos_sc)
    y = _gmm(be, x_perm, we, E, d)
    y0, y1 = _make_combine(N, d, nk_pad)(y, pos_sc)
    out = _final(x, ws, y0, y1, w01)
    return out.reshape(B, S, d), aux[0, 0]
